# Initial kernel scaffold; baseline (speedup 1.0000x reference)
import jax
import jax.numpy as jnp
from jax.experimental import pallas as pl


def kernel(x, batch):
    raise NotImplementedError("write your pallas kernel here")


# speedup vs baseline: 8.4078x; 8.4078x over previous
import functools

import jax
import jax.numpy as jnp
from jax import lax
from jax.experimental import pallas as pl
from jax.experimental.pallas import tpu as pltpu
from jax.experimental.pallas import tpu_sc as plsc

N_NODES = 100000
D_FEAT = 128
NUM_SEGMENTS = 512

NC = 2
NS = 16
NW = NC * NS
ROWS_PER_W = N_NODES // NW
CHUNK = 125
NCHUNK = ROWS_PER_W // CHUNK
STRIPE = NUM_SEGMENTS // NS

_mesh = plsc.VectorSubcoreMesh(core_axis_name="c", subcore_axis_name="s")


@functools.partial(
    pl.kernel,
    out_type=jax.ShapeDtypeStruct((NC, NUM_SEGMENTS, D_FEAT), jnp.float32),
    mesh=_mesh,
    scratch_types=[
        pltpu.VMEM((NCHUNK, CHUNK), jnp.int32),
        pltpu.VMEM((CHUNK, D_FEAT), jnp.float32),
        pltpu.VMEM((CHUNK, D_FEAT), jnp.float32),
        pltpu.VMEM((STRIPE, D_FEAT), jnp.float32),
        pltpu.VMEM_SHARED((NUM_SEGMENTS, D_FEAT), jnp.float32),
        pltpu.SemaphoreType.DMA,
        pltpu.SemaphoreType.DMA,
    ],
    compiler_params=pltpu.CompilerParams(use_tc_tiling_on_sc=False),
)
def _sc_segment_sum(x_hbm, ids_hbm, out_hbm, ids_v, buf0, buf1, sbuf,
                    acc_sh, sem0, sem1):
    c = lax.axis_index("c")
    s = lax.axis_index("s")
    wid = c * NS + s

    zeros16 = jnp.zeros((16,), jnp.float32)
    for r in range(STRIPE):
        for k in range(D_FEAT // 16):
            sbuf[r, pl.ds(k * 16, 16)] = zeros16
    pltpu.sync_copy(sbuf, acc_sh.at[pl.ds(s * STRIPE, STRIPE)])

    pltpu.sync_copy(ids_hbm.at[wid], ids_v)
    plsc.subcore_barrier()

    bufs = (buf0, buf1)
    sems = (sem0, sem1)
    base = wid * ROWS_PER_W
    cps = [None, None]
    cps[0] = pltpu.async_copy(x_hbm.at[pl.ds(base, CHUNK)], buf0, sem0)
    for j in range(NCHUNK):
        p = j % 2
        cps[p].wait()
        if j + 1 < NCHUNK:
            q = (j + 1) % 2
            cps[q] = pltpu.async_copy(
                x_hbm.at[pl.ds(base + (j + 1) * CHUNK, CHUNK)], bufs[q],
                sems[q])
        pltpu.sync_copy(bufs[p], acc_sh.at[ids_v.at[j]], add=True)

    plsc.subcore_barrier()

    pltpu.sync_copy(acc_sh.at[pl.ds(s * STRIPE, STRIPE)], sbuf)
    pltpu.sync_copy(sbuf, out_hbm.at[c, pl.ds(s * STRIPE, STRIPE)])


def _combine_body(a_ref, b_ref, o_ref):
    o_ref[...] = a_ref[...] + b_ref[...]


_combine = pl.pallas_call(
    _combine_body,
    out_shape=jax.ShapeDtypeStruct((NUM_SEGMENTS, D_FEAT), jnp.float32),
)


def kernel(x, batch):
    ids = batch.astype(jnp.int32).reshape(NW, NCHUNK, CHUNK)
    partials = _sc_segment_sum(x, ids)
    return _combine(partials[0], partials[1])
